# baseline (speedup 1.0000x reference)
import jax
import jax.numpy as jnp
from jax.experimental import pallas as pl
from jax.experimental.pallas import tpu as pltpu

T = 2048
D = 1024
F = 2048
E = 8
TOPK = 2
BT = 512
BF = 1024


def _moe_body(x_ref, g_ref, w1_ref, w3_ref, w2_ref, o_ref):
    e = pl.program_id(0)
    f = pl.program_id(1)
    t = pl.program_id(2)
    x = x_ref[pl.ds(t * BT, BT), :]

    logits = jnp.dot(x, g_ref[...], preferred_element_type=jnp.float32)
    m = jnp.max(logits, axis=1, keepdims=True)
    ex = jnp.exp(logits - m)
    p = ex / jnp.sum(ex, axis=1, keepdims=True)

    idx = jax.lax.broadcasted_iota(jnp.int32, (BT, E), 1)
    w1v = jnp.max(p, axis=1, keepdims=True)
    i1 = jnp.min(jnp.where(p == w1v, idx, E), axis=1, keepdims=True)
    p2 = jnp.where(idx == i1, -1.0, p)
    w2v = jnp.max(p2, axis=1, keepdims=True)
    i2 = jnp.min(jnp.where(p2 == w2v, idx, E), axis=1, keepdims=True)
    denom = w1v + w2v
    ew = jnp.where(i1 == e, w1v, jnp.where(i2 == e, w2v, 0.0)) / denom

    xb = x.astype(jnp.bfloat16)
    a = jnp.dot(xb, w1_ref[0], preferred_element_type=jnp.float32)
    b = jnp.dot(xb, w3_ref[0], preferred_element_type=jnp.float32)
    h = ((a * jax.lax.logistic(a)) * b).astype(jnp.bfloat16)
    y = jnp.dot(h, w2_ref[0], preferred_element_type=jnp.float32)

    @pl.when((e == 0) & (f == 0))
    def _():
        o_ref[pl.ds(t * BT, BT), :] = jnp.zeros((BT, D), jnp.float32)

    o_ref[pl.ds(t * BT, BT), :] += ew * y


@jax.jit
def kernel(hidden_states, gate_w, w1, w2, w3):
    w1 = w1.astype(jnp.bfloat16)
    w2 = w2.astype(jnp.bfloat16)
    w3 = w3.astype(jnp.bfloat16)
    grid = (E, F // BF, T // BT)
    return pl.pallas_call(
        _moe_body,
        grid=grid,
        in_specs=[
            pl.BlockSpec((T, D), lambda e, f, t: (0, 0)),
            pl.BlockSpec((D, E), lambda e, f, t: (0, 0)),
            pl.BlockSpec((1, D, BF), lambda e, f, t: (e, 0, f)),
            pl.BlockSpec((1, D, BF), lambda e, f, t: (e, 0, f)),
            pl.BlockSpec((1, BF, D), lambda e, f, t: (e, f, 0)),
        ],
        out_specs=pl.BlockSpec((T, D), lambda e, f, t: (0, 0)),
        out_shape=jax.ShapeDtypeStruct((T, D), jnp.float32),
        compiler_params=pltpu.CompilerParams(
            dimension_semantics=("arbitrary", "arbitrary", "arbitrary"),
        ),
    )(hidden_states, gate_w, w1, w3, w2)


# speedup vs baseline: 1.2907x; 1.2907x over previous
import jax
import jax.numpy as jnp
from jax.experimental import pallas as pl
from jax.experimental.pallas import tpu as pltpu

T = 2048
D = 1024
F = 2048
E = 8
TOPK = 2
BT = 512
BF = 1024


def _moe_body(x_ref, g_ref, w1_ref, w3_ref, w2_ref, o_ref):
    e = pl.program_id(0)
    f = pl.program_id(1)
    t = pl.program_id(2)
    x = x_ref[pl.ds(t * BT, BT), :]

    logits = jnp.dot(x, g_ref[...], preferred_element_type=jnp.float32)
    m = jnp.max(logits, axis=1, keepdims=True)
    ex = jnp.exp(logits - m)
    p = ex / jnp.sum(ex, axis=1, keepdims=True)

    idx = jax.lax.broadcasted_iota(jnp.int32, (BT, E), 1)
    w1v = jnp.max(p, axis=1, keepdims=True)
    i1 = jnp.min(jnp.where(p == w1v, idx, E), axis=1, keepdims=True)
    p2 = jnp.where(idx == i1, -1.0, p)
    w2v = jnp.max(p2, axis=1, keepdims=True)
    i2 = jnp.min(jnp.where(p2 == w2v, idx, E), axis=1, keepdims=True)
    denom = w1v + w2v
    ew = jnp.where(i1 == e, w1v, jnp.where(i2 == e, w2v, 0.0)) / denom

    xb = x.astype(jnp.bfloat16)
    w1b = w1_ref[0].astype(jnp.bfloat16)
    w3b = w3_ref[0].astype(jnp.bfloat16)
    w2b = w2_ref[0].astype(jnp.bfloat16)
    a = jnp.dot(xb, w1b, preferred_element_type=jnp.float32)
    b = jnp.dot(xb, w3b, preferred_element_type=jnp.float32)
    h = ((a * jax.lax.logistic(a)) * b).astype(jnp.bfloat16)
    y = jnp.dot(h, w2b, preferred_element_type=jnp.float32)

    @pl.when((e == 0) & (f == 0))
    def _():
        o_ref[pl.ds(t * BT, BT), :] = jnp.zeros((BT, D), jnp.float32)

    o_ref[pl.ds(t * BT, BT), :] += ew * y


@jax.jit
def kernel(hidden_states, gate_w, w1, w2, w3):
    grid = (E, F // BF, T // BT)
    return pl.pallas_call(
        _moe_body,
        grid=grid,
        in_specs=[
            pl.BlockSpec((T, D), lambda e, f, t: (0, 0)),
            pl.BlockSpec((D, E), lambda e, f, t: (0, 0)),
            pl.BlockSpec((1, D, BF), lambda e, f, t: (e, 0, f)),
            pl.BlockSpec((1, D, BF), lambda e, f, t: (e, 0, f)),
            pl.BlockSpec((1, BF, D), lambda e, f, t: (e, f, 0)),
        ],
        out_specs=pl.BlockSpec((T, D), lambda e, f, t: (0, 0)),
        out_shape=jax.ShapeDtypeStruct((T, D), jnp.float32),
        compiler_params=pltpu.CompilerParams(
            dimension_semantics=("arbitrary", "arbitrary", "arbitrary"),
        ),
    )(hidden_states, gate_w, w1, w3, w2)


# speedup vs baseline: 1.7544x; 1.3592x over previous
import jax
import jax.numpy as jnp
from jax import lax
from jax.experimental import pallas as pl
from jax.experimental.pallas import tpu as pltpu
from jax.experimental.pallas import tpu_sc as plsc

T = 2048
D = 1024
F = 2048
E = 8
TOPK = 2
BM = 256
NPAD = T * TOPK + E * BM
NB = NPAD // BM
NBP = 32
QW = 128

NW = 32
TPW = T // NW


def _route_body(x_ref, g_ref, s1_ref, s2_ref, q1_ref, q2_ref, blk_ref):
    x = x_ref[...]
    logits = jnp.dot(x, g_ref[...], preferred_element_type=jnp.float32)
    m = jnp.max(logits, axis=1, keepdims=True)
    ex = jnp.exp(logits - m)
    p = ex / jnp.sum(ex, axis=1, keepdims=True)

    idx = jax.lax.broadcasted_iota(jnp.int32, (T, E), 1)
    w1v = jnp.max(p, axis=1, keepdims=True)
    i1 = jnp.min(jnp.where(p == w1v, idx, E), axis=1, keepdims=True)
    p2 = jnp.where(idx == i1, -1.0, p)
    w2v = jnp.max(p2, axis=1, keepdims=True)
    i2 = jnp.min(jnp.where(p2 == w2v, idx, E), axis=1, keepdims=True)
    denom = w1v + w2v

    oh1 = (idx == i1).astype(jnp.float32)
    oh2 = (idx == i2).astype(jnp.float32)
    M = oh1 + oh2

    r = jax.lax.broadcasted_iota(jnp.int32, (128, 128), 0)
    c = jax.lax.broadcasted_iota(jnp.int32, (128, 128), 1)
    tril = (c < r).astype(jnp.float32)
    carry = jnp.zeros((1, E), jnp.float32)
    blocks = []
    for i in range(T // 128):
        Mi = M[i * 128:(i + 1) * 128, :]
        pb = jnp.dot(tril, Mi, preferred_element_type=jnp.float32)
        blocks.append(pb + carry)
        carry = carry + jnp.sum(Mi, axis=0, keepdims=True)
    gpref = jnp.concatenate(blocks, axis=0)
    counts = carry

    pcount = jnp.floor((counts + (BM - 1)) / BM) * BM
    e0 = jax.lax.broadcasted_iota(jnp.int32, (E, E), 0)
    e1 = jax.lax.broadcasted_iota(jnp.int32, (E, E), 1)
    S = (e0 < e1).astype(jnp.float32)
    poff = jnp.dot(pcount, S, preferred_element_type=jnp.float32)
    pcum = poff + pcount

    slot = poff + gpref
    s1_ref[...] = jnp.sum(oh1 * slot, axis=1, keepdims=True)
    s2_ref[...] = jnp.sum(oh2 * slot, axis=1, keepdims=True)
    q1_ref[...] = jnp.broadcast_to(w1v / denom, (T, QW))
    q2_ref[...] = jnp.broadcast_to(w2v / denom, (T, QW))

    bi = jax.lax.broadcasted_iota(jnp.int32, (NBP, E), 0).astype(jnp.float32) * BM
    bmask = (jnp.broadcast_to(pcum, (NBP, E)) <= bi).astype(jnp.float32)
    bexp = jnp.minimum(jnp.sum(bmask, axis=1, keepdims=True), float(E - 1))
    blk_ref[...] = jnp.broadcast_to(bexp, (NBP, E))


def _route(x, gate_w):
    out_shapes = (
        jax.ShapeDtypeStruct((T, 1), jnp.float32),
        jax.ShapeDtypeStruct((T, 1), jnp.float32),
        jax.ShapeDtypeStruct((T, QW), jnp.float32),
        jax.ShapeDtypeStruct((T, QW), jnp.float32),
        jax.ShapeDtypeStruct((NBP, E), jnp.float32),
    )
    return pl.pallas_call(_route_body, out_shape=out_shapes)(x, gate_w)


def _dispatch(x, s1, s2, q1, q2):
    mesh = plsc.VectorSubcoreMesh(core_axis_name="c", subcore_axis_name="s")

    def body(x_hbm, s1_hbm, s2_hbm, q1_hbm, q2_hbm, xs_hbm, qs_hbm,
             rows_v, q1_v, q2_v, i1_v, i2_v, sem):
        wid = lax.axis_index("s") * 2 + lax.axis_index("c")
        base = wid * TPW
        pltpu.sync_copy(x_hbm.at[pl.ds(base, TPW)], rows_v)
        pltpu.sync_copy(s1_hbm.at[pl.ds(base, TPW)], i1_v)
        pltpu.sync_copy(s2_hbm.at[pl.ds(base, TPW)], i2_v)
        pltpu.sync_copy(q1_hbm.at[pl.ds(base, TPW)], q1_v)
        pltpu.sync_copy(q2_hbm.at[pl.ds(base, TPW)], q2_v)
        pltpu.async_copy(rows_v, xs_hbm.at[i1_v], sem).wait()
        pltpu.async_copy(rows_v, xs_hbm.at[i2_v], sem).wait()
        pltpu.async_copy(q1_v, qs_hbm.at[i1_v], sem).wait()
        pltpu.async_copy(q2_v, qs_hbm.at[i2_v], sem).wait()

    k = pl.kernel(
        body,
        mesh=mesh,
        out_type=(
            jax.ShapeDtypeStruct((NPAD, D), jnp.float32),
            jax.ShapeDtypeStruct((NPAD, QW), jnp.float32),
        ),
        scratch_types=[
            pltpu.VMEM((TPW, D), jnp.float32),
            pltpu.VMEM((TPW, QW), jnp.float32),
            pltpu.VMEM((TPW, QW), jnp.float32),
            pltpu.VMEM((TPW,), jnp.int32),
            pltpu.VMEM((TPW,), jnp.int32),
            pltpu.SemaphoreType.DMA,
        ],
    )
    return k(x, s1, s2, q1, q2)


def _ffn_body(bmap_ref, xs_ref, qs_ref, w1_ref, w3_ref, w2_ref, o_ref):
    xb = xs_ref[...].astype(jnp.bfloat16)
    w1b = w1_ref[0].astype(jnp.bfloat16)
    w3b = w3_ref[0].astype(jnp.bfloat16)
    w2b = w2_ref[0].astype(jnp.bfloat16)
    a = jnp.dot(xb, w1b, preferred_element_type=jnp.float32)
    b = jnp.dot(xb, w3b, preferred_element_type=jnp.float32)
    h = ((a * jax.lax.logistic(a)) * b).astype(jnp.bfloat16)
    y = jnp.dot(h, w2b, preferred_element_type=jnp.float32)
    o_ref[...] = y * qs_ref[...][:, 0:1]


def _ffn(bmap, xs, qs, w1, w2, w3):
    grid_spec = pltpu.PrefetchScalarGridSpec(
        num_scalar_prefetch=1,
        grid=(NB,),
        in_specs=[
            pl.BlockSpec((BM, D), lambda b, m: (b, 0)),
            pl.BlockSpec((BM, QW), lambda b, m: (b, 0)),
            pl.BlockSpec((1, D, F), lambda b, m: (m[b], 0, 0)),
            pl.BlockSpec((1, D, F), lambda b, m: (m[b], 0, 0)),
            pl.BlockSpec((1, F, D), lambda b, m: (m[b], 0, 0)),
        ],
        out_specs=pl.BlockSpec((BM, D), lambda b, m: (b, 0)),
    )
    return pl.pallas_call(
        _ffn_body,
        grid_spec=grid_spec,
        out_shape=jax.ShapeDtypeStruct((NPAD, D), jnp.float32),
        compiler_params=pltpu.CompilerParams(
            dimension_semantics=("arbitrary",),
        ),
    )(bmap, xs, qs, w1, w3, w2)


def _combine(ys, s1, s2):
    mesh = plsc.VectorSubcoreMesh(core_axis_name="c", subcore_axis_name="s")
    HT = 32

    def body(ys_hbm, s1_hbm, s2_hbm, out_hbm, r1_v, r2_v, o_v, i1_v, i2_v, sem):
        wid = lax.axis_index("s") * 2 + lax.axis_index("c")

        def half(h, _):
            base = wid * TPW + h * HT
            pltpu.sync_copy(s1_hbm.at[pl.ds(base, HT)], i1_v)
            pltpu.sync_copy(s2_hbm.at[pl.ds(base, HT)], i2_v)
            pltpu.async_copy(ys_hbm.at[i1_v], r1_v, sem).wait()
            pltpu.async_copy(ys_hbm.at[i2_v], r2_v, sem).wait()

            def tok(t, _):
                def col(j, _):
                    sl = pl.ds(j * 16, 16)
                    o_v[t, sl] = r1_v[t, sl] + r2_v[t, sl]
                    return 0

                return lax.fori_loop(0, D // 16, col, 0)

            lax.fori_loop(0, HT, tok, 0)
            pltpu.sync_copy(o_v, out_hbm.at[pl.ds(base, HT)])
            return 0

        lax.fori_loop(0, 2, half, 0)

    k = pl.kernel(
        body,
        mesh=mesh,
        out_type=jax.ShapeDtypeStruct((T, D), jnp.float32),
        scratch_types=[
            pltpu.VMEM((HT, D), jnp.float32),
            pltpu.VMEM((HT, D), jnp.float32),
            pltpu.VMEM((HT, D), jnp.float32),
            pltpu.VMEM((HT,), jnp.int32),
            pltpu.VMEM((HT,), jnp.int32),
            pltpu.SemaphoreType.DMA,
        ],
    )
    return k(ys, s1, s2)


@jax.jit
def kernel(hidden_states, gate_w, w1, w2, w3):
    s1f, s2f, q1f, q2f, blkf = _route(hidden_states, gate_w)
    s1 = s1f.reshape(T).astype(jnp.int32)
    s2 = s2f.reshape(T).astype(jnp.int32)
    bmap = blkf[:NB, 0].astype(jnp.int32)
    xs, qs = _dispatch(hidden_states, s1, s2, q1f, q2f)
    ys = _ffn(bmap, xs, qs, w1, w2, w3)
    return _combine(ys, s1, s2)


# speedup vs baseline: 1.8399x; 1.0488x over previous
import jax
import jax.numpy as jnp
from jax import lax
from jax.experimental import pallas as pl
from jax.experimental.pallas import tpu as pltpu
from jax.experimental.pallas import tpu_sc as plsc

T = 2048
D = 1024
F = 2048
E = 8
TOPK = 2
BM = 256
NPAD = T * TOPK + E * BM
NB = NPAD // BM
NBP = 32
QW = 128

NW = 32
TPW = T // NW


def _route_body(x_ref, g_ref, s1_ref, s2_ref, q1_ref, q2_ref, blk_ref):
    x = x_ref[...]
    logits = jnp.dot(x, g_ref[...], preferred_element_type=jnp.float32)
    m = jnp.max(logits, axis=1, keepdims=True)
    ex = jnp.exp(logits - m)
    p = ex / jnp.sum(ex, axis=1, keepdims=True)

    idx = jax.lax.broadcasted_iota(jnp.int32, (T, E), 1)
    w1v = jnp.max(p, axis=1, keepdims=True)
    i1 = jnp.min(jnp.where(p == w1v, idx, E), axis=1, keepdims=True)
    p2 = jnp.where(idx == i1, -1.0, p)
    w2v = jnp.max(p2, axis=1, keepdims=True)
    i2 = jnp.min(jnp.where(p2 == w2v, idx, E), axis=1, keepdims=True)
    denom = w1v + w2v

    oh1 = (idx == i1).astype(jnp.float32)
    oh2 = (idx == i2).astype(jnp.float32)
    M = oh1 + oh2

    r = jax.lax.broadcasted_iota(jnp.int32, (128, 128), 0)
    c = jax.lax.broadcasted_iota(jnp.int32, (128, 128), 1)
    tril = (c < r).astype(jnp.float32)
    carry = jnp.zeros((1, E), jnp.float32)
    blocks = []
    for i in range(T // 128):
        Mi = M[i * 128:(i + 1) * 128, :]
        pb = jnp.dot(tril, Mi, preferred_element_type=jnp.float32)
        blocks.append(pb + carry)
        carry = carry + jnp.sum(Mi, axis=0, keepdims=True)
    gpref = jnp.concatenate(blocks, axis=0)
    counts = carry

    pcount = jnp.floor((counts + (BM - 1)) / BM) * BM
    e0 = jax.lax.broadcasted_iota(jnp.int32, (E, E), 0)
    e1 = jax.lax.broadcasted_iota(jnp.int32, (E, E), 1)
    S = (e0 < e1).astype(jnp.float32)
    poff = jnp.dot(pcount, S, preferred_element_type=jnp.float32)
    pcum = poff + pcount

    slot = poff + gpref
    s1_ref[...] = jnp.sum(oh1 * slot, axis=1, keepdims=True)
    s2_ref[...] = jnp.sum(oh2 * slot, axis=1, keepdims=True)
    q1_ref[...] = jnp.broadcast_to(w1v / denom, (T, QW))
    q2_ref[...] = jnp.broadcast_to(w2v / denom, (T, QW))

    bi = jax.lax.broadcasted_iota(jnp.int32, (NBP, E), 0).astype(jnp.float32) * BM
    bmask = (jnp.broadcast_to(pcum, (NBP, E)) <= bi).astype(jnp.float32)
    bexp = jnp.minimum(jnp.sum(bmask, axis=1, keepdims=True), float(E - 1))
    blk_ref[...] = jnp.broadcast_to(bexp, (NBP, E))


def _route(x, gate_w):
    out_shapes = (
        jax.ShapeDtypeStruct((T, 1), jnp.float32),
        jax.ShapeDtypeStruct((T, 1), jnp.float32),
        jax.ShapeDtypeStruct((T, QW), jnp.float32),
        jax.ShapeDtypeStruct((T, QW), jnp.float32),
        jax.ShapeDtypeStruct((NBP, E), jnp.float32),
    )
    return pl.pallas_call(_route_body, out_shape=out_shapes)(x, gate_w)


def _dispatch(x, s1, s2, q1, q2):
    mesh = plsc.VectorSubcoreMesh(core_axis_name="c", subcore_axis_name="s")

    def body(x_hbm, s1_hbm, s2_hbm, q1_hbm, q2_hbm, xs_hbm, qs_hbm,
             rows_v, q1_v, q2_v, i1_v, i2_v, sem):
        wid = lax.axis_index("s") * 2 + lax.axis_index("c")
        base = wid * TPW
        pltpu.sync_copy(x_hbm.at[pl.ds(base, TPW)], rows_v)
        pltpu.sync_copy(s1_hbm.at[pl.ds(base, TPW)], i1_v)
        pltpu.sync_copy(s2_hbm.at[pl.ds(base, TPW)], i2_v)
        pltpu.sync_copy(q1_hbm.at[pl.ds(base, TPW)], q1_v)
        pltpu.sync_copy(q2_hbm.at[pl.ds(base, TPW)], q2_v)
        c1 = pltpu.async_copy(rows_v, xs_hbm.at[i1_v], sem)
        c2 = pltpu.async_copy(rows_v, xs_hbm.at[i2_v], sem)
        c3 = pltpu.async_copy(q1_v, qs_hbm.at[i1_v], sem)
        c4 = pltpu.async_copy(q2_v, qs_hbm.at[i2_v], sem)
        c1.wait()
        c2.wait()
        c3.wait()
        c4.wait()

    k = pl.kernel(
        body,
        mesh=mesh,
        out_type=(
            jax.ShapeDtypeStruct((NPAD, D), jnp.float32),
            jax.ShapeDtypeStruct((NPAD, QW), jnp.float32),
        ),
        scratch_types=[
            pltpu.VMEM((TPW, D), jnp.float32),
            pltpu.VMEM((TPW, QW), jnp.float32),
            pltpu.VMEM((TPW, QW), jnp.float32),
            pltpu.VMEM((TPW,), jnp.int32),
            pltpu.VMEM((TPW,), jnp.int32),
            pltpu.SemaphoreType.DMA,
        ],
    )
    return k(x, s1, s2, q1, q2)


def _ffn_body(bmap_ref, xs_ref, qs_ref, w1_ref, w3_ref, w2_ref, o_ref):
    xb = xs_ref[...].astype(jnp.bfloat16)
    w1b = w1_ref[0].astype(jnp.bfloat16)
    w3b = w3_ref[0].astype(jnp.bfloat16)
    w2b = w2_ref[0].astype(jnp.bfloat16)
    a = jnp.dot(xb, w1b, preferred_element_type=jnp.float32)
    b = jnp.dot(xb, w3b, preferred_element_type=jnp.float32)
    h = ((a * jax.lax.logistic(a)) * b).astype(jnp.bfloat16)
    y = jnp.dot(h, w2b, preferred_element_type=jnp.float32)
    o_ref[...] = y * qs_ref[...][:, 0:1]


def _ffn(bmap, xs, qs, w1, w2, w3):
    grid_spec = pltpu.PrefetchScalarGridSpec(
        num_scalar_prefetch=1,
        grid=(NB,),
        in_specs=[
            pl.BlockSpec((BM, D), lambda b, m: (b, 0)),
            pl.BlockSpec((BM, QW), lambda b, m: (b, 0)),
            pl.BlockSpec((1, D, F), lambda b, m: (m[b], 0, 0)),
            pl.BlockSpec((1, D, F), lambda b, m: (m[b], 0, 0)),
            pl.BlockSpec((1, F, D), lambda b, m: (m[b], 0, 0)),
        ],
        out_specs=pl.BlockSpec((BM, D), lambda b, m: (b, 0)),
    )
    return pl.pallas_call(
        _ffn_body,
        grid_spec=grid_spec,
        out_shape=jax.ShapeDtypeStruct((NPAD, D), jnp.float32),
        compiler_params=pltpu.CompilerParams(
            dimension_semantics=("arbitrary",),
        ),
    )(bmap, xs, qs, w1, w3, w2)


def _combine(ys, s1, s2):
    mesh = plsc.VectorSubcoreMesh(core_axis_name="c", subcore_axis_name="s")
    HT = 32

    def body(ys_hbm, s1_hbm, s2_hbm, out_hbm, r1_v, r2_v, o_v, i1_v, i2_v, sem):
        wid = lax.axis_index("s") * 2 + lax.axis_index("c")

        def half(h, _):
            base = wid * TPW + h * HT
            pltpu.sync_copy(s1_hbm.at[pl.ds(base, HT)], i1_v)
            pltpu.sync_copy(s2_hbm.at[pl.ds(base, HT)], i2_v)
            g1 = pltpu.async_copy(ys_hbm.at[i1_v], r1_v, sem)
            g2 = pltpu.async_copy(ys_hbm.at[i2_v], r2_v, sem)
            g1.wait()
            g2.wait()

            def tok(t, _):
                for j in range(D // 16):
                    sl = pl.ds(j * 16, 16)
                    o_v[t, sl] = r1_v[t, sl] + r2_v[t, sl]
                return 0

            lax.fori_loop(0, HT, tok, 0)
            pltpu.sync_copy(o_v, out_hbm.at[pl.ds(base, HT)])
            return 0

        lax.fori_loop(0, 2, half, 0)

    k = pl.kernel(
        body,
        mesh=mesh,
        out_type=jax.ShapeDtypeStruct((T, D), jnp.float32),
        scratch_types=[
            pltpu.VMEM((HT, D), jnp.float32),
            pltpu.VMEM((HT, D), jnp.float32),
            pltpu.VMEM((HT, D), jnp.float32),
            pltpu.VMEM((HT,), jnp.int32),
            pltpu.VMEM((HT,), jnp.int32),
            pltpu.SemaphoreType.DMA,
        ],
    )
    return k(ys, s1, s2)


@jax.jit
def kernel(hidden_states, gate_w, w1, w2, w3):
    s1f, s2f, q1f, q2f, blkf = _route(hidden_states, gate_w)
    s1 = s1f.reshape(T).astype(jnp.int32)
    s2 = s2f.reshape(T).astype(jnp.int32)
    bmap = blkf[:NB, 0].astype(jnp.int32)
    xs, qs = _dispatch(hidden_states, s1, s2, q1f, q2f)
    ys = _ffn(bmap, xs, qs, w1, w2, w3)
    return _combine(ys, s1, s2)
